# baseline (speedup 1.0000x reference)
import functools

import jax
import jax.numpy as jnp
from jax import lax
from jax.experimental import pallas as pl
from jax.experimental.pallas import tpu as pltpu
from jax.experimental.pallas import tpu_sc as plsc

E_BLOCK = 4000
N_BLOCK = 2000

SC_CORES = 2
SC_SUBCORES = 16
NW = SC_CORES * SC_SUBCORES
S_BATCH = 125
S_K = 8
CHUNK = S_BATCH * S_K


def _bdot(x, w):
    return jnp.dot(x.astype(jnp.bfloat16), w.astype(jnp.bfloat16),
                   preferred_element_type=jnp.float32)


def _edge_body(x_ref, w0, b0, w1, b1, w2, b2, w3, b3, w4, b4, out_ref):
    x = x_ref[...]
    x = jnp.maximum(_bdot(x, w0[...]) + b0[...], 0.0)
    x = jnp.maximum(_bdot(x, w1[...]) + b1[...], 0.0)
    x = jnp.maximum(_bdot(x, w2[...]) + b2[...], 0.0)
    x = jnp.maximum(_bdot(x, w3[...]) + b3[...], 0.0)
    out_ref[...] = _bdot(x, w4[...]) + b4[...]


def _full_spec(shape):
    return pl.BlockSpec(shape, lambda i: (0,) * len(shape))


def _edge_mlp(x, Ws, bs):
    e, fin = x.shape
    fout = Ws[-1].shape[1]
    specs = [pl.BlockSpec((E_BLOCK, fin), lambda i: (i, 0))]
    args = [x]
    for w, b in zip(Ws, bs):
        specs.append(_full_spec(w.shape))
        args.append(w)
        b2 = b.reshape(1, -1)
        specs.append(_full_spec(b2.shape))
        args.append(b2)
    return pl.pallas_call(
        _edge_body,
        grid=(e // E_BLOCK,),
        in_specs=specs,
        out_specs=pl.BlockSpec((E_BLOCK, fout), lambda i: (i, 0)),
        out_shape=jax.ShapeDtypeStruct((e, fout), jnp.float32),
    )(*args)


def _node_body(dyn_ref, p0_ref, p1_ref, w0a, w0b, b0, w1, b1, w2, b2, w3, b3,
               w4, b4, out_ref):
    h = _bdot(dyn_ref[...], w0a[...])
    h = h + _bdot(p0_ref[...] + p1_ref[...], w0b[...])
    h = jnp.maximum(h + b0[...], 0.0)
    h = jnp.maximum(_bdot(h, w1[...]) + b1[...], 0.0)
    h = jnp.maximum(_bdot(h, w2[...]) + b2[...], 0.0)
    h = jnp.maximum(_bdot(h, w3[...]) + b3[...], 0.0)
    out_ref[...] = _bdot(h, w4[...]) + b4[...]


def _node_mlp(dyn, p0, p1, Ws, bs):
    n, fdyn = dyn.shape
    fagg = p0.shape[1]
    fout = Ws[-1].shape[1]
    w0a = Ws[0][:fdyn]
    w0b = Ws[0][fdyn:]
    specs = [
        pl.BlockSpec((N_BLOCK, fdyn), lambda i: (i, 0)),
        pl.BlockSpec((N_BLOCK, fagg), lambda i: (i, 0)),
        pl.BlockSpec((N_BLOCK, fagg), lambda i: (i, 0)),
        _full_spec(w0a.shape),
        _full_spec(w0b.shape),
    ]
    args = [dyn, p0, p1, w0a, w0b]
    b2 = bs[0].reshape(1, -1)
    specs.append(_full_spec(b2.shape))
    args.append(b2)
    for w, b in zip(Ws[1:], bs[1:]):
        specs.append(_full_spec(w.shape))
        args.append(w)
        b2 = b.reshape(1, -1)
        specs.append(_full_spec(b2.shape))
        args.append(b2)
    return pl.pallas_call(
        _node_body,
        grid=(n // N_BLOCK,),
        in_specs=specs,
        out_specs=pl.BlockSpec((N_BLOCK, fout), lambda i: (i, 0)),
        out_shape=jax.ShapeDtypeStruct((n, fout), jnp.float32),
    )(*args)


def _sc_scatter(msg, recv2, n):
    e = msg.shape[0]
    rounds = e // (NW * CHUNK)
    rows_per_worker = rounds * S_K
    align = 8 * SC_SUBCORES
    n_pad = ((n + align - 1) // align) * align
    stripe = n_pad // SC_SUBCORES

    zstripe = jnp.zeros((stripe, 16), jnp.float32)
    mesh = plsc.VectorSubcoreMesh(core_axis_name="c", subcore_axis_name="s")

    @functools.partial(
        pl.kernel,
        mesh=mesh,
        out_type=jax.ShapeDtypeStruct((SC_CORES, n_pad, 16), jnp.float32),
        scratch_types=[
            pltpu.VMEM((S_K, S_BATCH), jnp.int32),
            pltpu.VMEM((CHUNK, 16), jnp.float32),
            pltpu.VMEM_SHARED((n_pad, 16), jnp.float32),
        ],
        compiler_params=pltpu.CompilerParams(use_tc_tiling_on_sc=False),
    )
    def k(msg_hbm, recv_hbm, z_hbm, out_hbm, idx_v, msg_v, agg_sh):
        c = lax.axis_index("c")
        s = lax.axis_index("s")
        w = s * SC_CORES + c

        pltpu.sync_copy(z_hbm, agg_sh.at[pl.ds(s * stripe, stripe)])
        plsc.subcore_barrier()

        @pl.loop(0, rounds)
        def _(r):
            row0 = w * rows_per_worker + r * S_K
            pltpu.sync_copy(recv_hbm.at[pl.ds(row0, S_K)], idx_v)
            pltpu.sync_copy(msg_hbm.at[pl.ds(row0 * S_BATCH, CHUNK)], msg_v)
            for j in range(S_K):
                pltpu.sync_copy(
                    msg_v.at[pl.ds(j * S_BATCH, S_BATCH)],
                    agg_sh.at[idx_v.at[j]],
                    add=True,
                )

        plsc.subcore_barrier()
        pltpu.sync_copy(agg_sh.at[pl.ds(s * stripe, stripe)],
                        out_hbm.at[c, pl.ds(s * stripe, stripe)])

    return k(msg, recv2, zstripe)[:, :n, :]


def kernel(dyn, rel, send, recv,
           frel_W0, frel_b0, frel_W1, frel_b1, frel_W2, frel_b2, frel_W3,
           frel_b3, frel_W4, frel_b4,
           fdyn_W0, fdyn_b0, fdyn_W1, fdyn_b1, fdyn_W2, fdyn_b2, fdyn_W3,
           fdyn_b3, fdyn_W4, fdyn_b4):
    frel_Ws = [frel_W0, frel_W1, frel_W2, frel_W3, frel_W4]
    frel_bs = [frel_b0, frel_b1, frel_b2, frel_b3, frel_b4]
    fdyn_Ws = [fdyn_W0, fdyn_W1, fdyn_W2, fdyn_W3, fdyn_W4]
    fdyn_bs = [fdyn_b0, fdyn_b1, fdyn_b2, fdyn_b3, fdyn_b4]
    b, n, _ = dyn.shape
    e = rel.shape[1]

    msg = _edge_mlp(rel.reshape(e, rel.shape[-1]), frel_Ws, frel_bs)

    recv2 = recv.reshape(e // S_BATCH, S_BATCH)
    parts = _sc_scatter(msg, recv2, n)

    out = _node_mlp(dyn.reshape(n, dyn.shape[-1]), parts[0], parts[1],
                    fdyn_Ws, fdyn_bs)
    return out.reshape(b, n, out.shape[-1])


# speedup vs baseline: 1.0372x; 1.0372x over previous
import functools

import jax
import jax.numpy as jnp
from jax import lax
from jax.experimental import pallas as pl
from jax.experimental.pallas import tpu as pltpu
from jax.experimental.pallas import tpu_sc as plsc

E_BLOCK = 4000
N_BLOCK = 2000
K_CH = 2

SC_CORES = 2
SC_SUBCORES = 16
NW = SC_CORES * SC_SUBCORES
S_BATCH = 125
S_K = 4
CHUNK = S_BATCH * S_K


def _bdot(x, w):
    return jnp.dot(x, w, preferred_element_type=jnp.float32)


def _edge_body(x_ref, w0, b0, w1, b1, w2, b2, w3, b3, w4, b4, out_ref):
    x = x_ref[...]
    x = jnp.maximum(_bdot(x, w0[...]) + b0[...], 0.0)
    x = jnp.maximum(_bdot(x, w1[...]) + b1[...], 0.0)
    x = jnp.maximum(_bdot(x, w2[...]) + b2[...], 0.0)
    x = jnp.maximum(_bdot(x, w3[...]) + b3[...], 0.0)
    out_ref[...] = _bdot(x, w4[...]) + b4[...]


def _full_spec(shape):
    return pl.BlockSpec(shape, lambda i: (0,) * len(shape))


def _edge_mlp_chunk(x, Ws, bs, e_ch, block_off):
    fin = x.shape[1]
    fout = Ws[-1].shape[1]
    specs = [pl.BlockSpec((E_BLOCK, fin), lambda i: (i + block_off, 0))]
    args = [x]
    for w, b in zip(Ws, bs):
        specs.append(_full_spec(w.shape))
        args.append(w)
        b2 = b.reshape(1, -1)
        specs.append(_full_spec(b2.shape))
        args.append(b2)
    return pl.pallas_call(
        _edge_body,
        grid=(e_ch // E_BLOCK,),
        in_specs=specs,
        out_specs=pl.BlockSpec((E_BLOCK, fout), lambda i: (i, 0)),
        out_shape=jax.ShapeDtypeStruct((e_ch, fout), jnp.float32),
    )(*args)


def _node_mlp(dyn, parts, Ws, bs):
    n, fdyn = dyn.shape
    fagg = parts[0].shape[1]
    fout = Ws[-1].shape[1]
    w0a = Ws[0][:fdyn]
    w0b = Ws[0][fdyn:]
    nparts = len(parts)

    def body(*refs):
        dyn_ref = refs[0]
        prefs = refs[1:1 + nparts]
        (w0a_r, w0b_r, b0_r, w1_r, b1_r, w2_r, b2_r, w3_r, b3_r, w4_r,
         b4_r) = refs[1 + nparts:-1]
        out_ref = refs[-1]
        p = prefs[0][...]
        for pr in prefs[1:]:
            p = p + pr[...]
        h = _bdot(dyn_ref[...], w0a_r[...]) + _bdot(p, w0b_r[...])
        h = jnp.maximum(h + b0_r[...], 0.0)
        h = jnp.maximum(_bdot(h, w1_r[...]) + b1_r[...], 0.0)
        h = jnp.maximum(_bdot(h, w2_r[...]) + b2_r[...], 0.0)
        h = jnp.maximum(_bdot(h, w3_r[...]) + b3_r[...], 0.0)
        out_ref[...] = _bdot(h, w4_r[...]) + b4_r[...]

    specs = [pl.BlockSpec((N_BLOCK, fdyn), lambda i: (i, 0))]
    args = [dyn]
    for p in parts:
        specs.append(pl.BlockSpec((N_BLOCK, fagg), lambda i: (i, 0)))
        args.append(p)
    specs += [_full_spec(w0a.shape), _full_spec(w0b.shape)]
    args += [w0a, w0b]
    b2 = bs[0].reshape(1, -1)
    specs.append(_full_spec(b2.shape))
    args.append(b2)
    for w, b in zip(Ws[1:], bs[1:]):
        specs.append(_full_spec(w.shape))
        args.append(w)
        b2 = b.reshape(1, -1)
        specs.append(_full_spec(b2.shape))
        args.append(b2)
    return pl.pallas_call(
        body,
        grid=(n // N_BLOCK,),
        in_specs=specs,
        out_specs=pl.BlockSpec((N_BLOCK, fout), lambda i: (i, 0)),
        out_shape=jax.ShapeDtypeStruct((n, fout), jnp.float32),
    )(*args)


def _sc_scatter(msg, recv2, n, row_off):
    e_ch = msg.shape[0]
    rounds = e_ch // (NW * CHUNK)
    assert rounds % 2 == 0 and rounds >= 4
    rows_per_worker = rounds * S_K
    align = 8 * SC_SUBCORES
    n_pad = ((n + align - 1) // align) * align
    stripe = n_pad // SC_SUBCORES

    zstripe = jnp.zeros((stripe, 16), jnp.float32)
    mesh = plsc.VectorSubcoreMesh(core_axis_name="c", subcore_axis_name="s")

    @functools.partial(
        pl.kernel,
        mesh=mesh,
        out_type=jax.ShapeDtypeStruct((SC_CORES, n_pad, 16), jnp.float32),
        scratch_types=[
            pltpu.VMEM((2, S_K, S_BATCH), jnp.int32),
            pltpu.VMEM((2, CHUNK, 16), jnp.float32),
            pltpu.VMEM_SHARED((n_pad, 16), jnp.float32),
            pltpu.SemaphoreType.DMA,
            pltpu.SemaphoreType.DMA,
            pltpu.SemaphoreType.DMA,
            pltpu.SemaphoreType.DMA,
        ],
        compiler_params=pltpu.CompilerParams(use_tc_tiling_on_sc=False),
    )
    def k(msg_hbm, recv_hbm, z_hbm, out_hbm, idx_v, msg_v, agg_sh,
          si0, sm0, si1, sm1):
        c = lax.axis_index("c")
        s = lax.axis_index("s")
        w = s * SC_CORES + c
        base = w * rows_per_worker
        sems = ((si0, sm0), (si1, sm1))

        def issue(rr, p):
            r0 = base + rr * S_K
            pltpu.async_copy(recv_hbm.at[pl.ds(row_off + r0, S_K)],
                             idx_v.at[p], sems[p][0])
            pltpu.async_copy(msg_hbm.at[pl.ds(r0 * S_BATCH, CHUNK)],
                             msg_v.at[p], sems[p][1])

        def wait(p):
            pltpu.make_async_copy(recv_hbm.at[pl.ds(row_off, S_K)],
                                  idx_v.at[p], sems[p][0]).wait()
            pltpu.make_async_copy(msg_hbm.at[pl.ds(0, CHUNK)],
                                  msg_v.at[p], sems[p][1]).wait()

        def scatter(p):
            for j in range(S_K):
                pltpu.sync_copy(
                    msg_v.at[p, pl.ds(j * S_BATCH, S_BATCH)],
                    agg_sh.at[idx_v.at[p, j]],
                    add=True,
                )

        pltpu.sync_copy(z_hbm, agg_sh.at[pl.ds(s * stripe, stripe)])
        plsc.subcore_barrier()

        issue(0, 0)
        issue(1, 1)

        @pl.loop(0, rounds - 2, step=2)
        def _(r):
            wait(0)
            scatter(0)
            issue(r + 2, 0)
            wait(1)
            scatter(1)
            issue(r + 3, 1)

        wait(0)
        scatter(0)
        wait(1)
        scatter(1)

        plsc.subcore_barrier()
        pltpu.sync_copy(agg_sh.at[pl.ds(s * stripe, stripe)],
                        out_hbm.at[c, pl.ds(s * stripe, stripe)])

    return k(msg, recv2, zstripe)


def kernel(dyn, rel, send, recv,
           frel_W0, frel_b0, frel_W1, frel_b1, frel_W2, frel_b2, frel_W3,
           frel_b3, frel_W4, frel_b4,
           fdyn_W0, fdyn_b0, fdyn_W1, fdyn_b1, fdyn_W2, fdyn_b2, fdyn_W3,
           fdyn_b3, fdyn_W4, fdyn_b4):
    frel_Ws = [frel_W0, frel_W1, frel_W2, frel_W3, frel_W4]
    frel_bs = [frel_b0, frel_b1, frel_b2, frel_b3, frel_b4]
    fdyn_Ws = [fdyn_W0, fdyn_W1, fdyn_W2, fdyn_W3, fdyn_W4]
    fdyn_bs = [fdyn_b0, fdyn_b1, fdyn_b2, fdyn_b3, fdyn_b4]
    b, n, _ = dyn.shape
    e = rel.shape[1]
    e_ch = e // K_CH

    rel2 = rel.reshape(e, rel.shape[-1])
    recv2 = recv.reshape(e // S_BATCH, S_BATCH)

    parts = []
    for kc in range(K_CH):
        msg_k = _edge_mlp_chunk(rel2, frel_Ws, frel_bs, e_ch,
                                kc * (e_ch // E_BLOCK))
        pk = _sc_scatter(msg_k, recv2, n, kc * (e_ch // S_BATCH))
        parts += [pk[0], pk[1]]

    out = _node_mlp(dyn.reshape(n, dyn.shape[-1]), parts, fdyn_Ws, fdyn_bs)
    return out.reshape(b, n, out.shape[-1])


# speedup vs baseline: 1.1224x; 1.0822x over previous
import functools

import jax
import jax.numpy as jnp
from jax import lax
from jax.experimental import pallas as pl
from jax.experimental.pallas import tpu as pltpu
from jax.experimental.pallas import tpu_sc as plsc

E_BLOCK = 8000
N_BLOCK = 2000
K_CH = 2

SC_CORES = 2
SC_SUBCORES = 16
NW = SC_CORES * SC_SUBCORES
S_BATCH = 125
S_PAD = 128
S_K = 4
CHUNK = S_BATCH * S_K


def _bdot(x, w):
    return jnp.dot(x, w, preferred_element_type=jnp.float32)


def _edge_body(x_ref, w0, b0, w1, b1, w2, b2, w3, b3, w4, b4, out_ref):
    x = x_ref[...]
    x = jnp.maximum(_bdot(x, w0[...]) + b0[...], 0.0)
    x = jnp.maximum(_bdot(x, w1[...]) + b1[...], 0.0)
    x = jnp.maximum(_bdot(x, w2[...]) + b2[...], 0.0)
    x = jnp.maximum(_bdot(x, w3[...]) + b3[...], 0.0)
    out_ref[...] = _bdot(x, w4[...]) + b4[...]


def _full_spec(shape):
    return pl.BlockSpec(shape, lambda i: (0,) * len(shape))


def _edge_mlp_chunk(x, Ws, bs, e_ch, block_off):
    fin = x.shape[1]
    fout = Ws[-1].shape[1]
    specs = [pl.BlockSpec((E_BLOCK, fin), lambda i: (i + block_off, 0))]
    args = [x]
    for w, b in zip(Ws, bs):
        specs.append(_full_spec(w.shape))
        args.append(w)
        b2 = b.reshape(1, -1)
        specs.append(_full_spec(b2.shape))
        args.append(b2)
    return pl.pallas_call(
        _edge_body,
        grid=(e_ch // E_BLOCK,),
        in_specs=specs,
        out_specs=pl.BlockSpec((E_BLOCK, fout), lambda i: (i, 0)),
        out_shape=jax.ShapeDtypeStruct((e_ch, fout), jnp.float32),
    )(*args)


def _node_mlp(dyn, parts, Ws, bs):
    n, fdyn = dyn.shape
    fagg = parts[0].shape[1]
    fout = Ws[-1].shape[1]
    w0a = Ws[0][:fdyn]
    w0b = Ws[0][fdyn:]
    nparts = len(parts)

    def body(*refs):
        dyn_ref = refs[0]
        prefs = refs[1:1 + nparts]
        (w0a_r, w0b_r, b0_r, w1_r, b1_r, w2_r, b2_r, w3_r, b3_r, w4_r,
         b4_r) = refs[1 + nparts:-1]
        out_ref = refs[-1]
        p = prefs[0][...]
        for pr in prefs[1:]:
            p = p + pr[...]
        h = _bdot(dyn_ref[...], w0a_r[...]) + _bdot(p, w0b_r[...])
        h = jnp.maximum(h + b0_r[...], 0.0)
        h = jnp.maximum(_bdot(h, w1_r[...]) + b1_r[...], 0.0)
        h = jnp.maximum(_bdot(h, w2_r[...]) + b2_r[...], 0.0)
        h = jnp.maximum(_bdot(h, w3_r[...]) + b3_r[...], 0.0)
        out_ref[...] = _bdot(h, w4_r[...]) + b4_r[...]

    specs = [pl.BlockSpec((N_BLOCK, fdyn), lambda i: (i, 0))]
    args = [dyn]
    for p in parts:
        specs.append(pl.BlockSpec((N_BLOCK, fagg), lambda i: (i, 0)))
        args.append(p)
    specs += [_full_spec(w0a.shape), _full_spec(w0b.shape)]
    args += [w0a, w0b]
    b2 = bs[0].reshape(1, -1)
    specs.append(_full_spec(b2.shape))
    args.append(b2)
    for w, b in zip(Ws[1:], bs[1:]):
        specs.append(_full_spec(w.shape))
        args.append(w)
        b2 = b.reshape(1, -1)
        specs.append(_full_spec(b2.shape))
        args.append(b2)
    return pl.pallas_call(
        body,
        grid=(n // N_BLOCK,),
        in_specs=specs,
        out_specs=pl.BlockSpec((N_BLOCK, fout), lambda i: (i, 0)),
        out_shape=jax.ShapeDtypeStruct((n, fout), jnp.float32),
    )(*args)


def _sc_scatter(msg, recv2, n, row_off):
    e_ch = msg.shape[0]
    rounds = e_ch // (NW * CHUNK)
    assert rounds % 2 == 0 and rounds >= 4
    rows_per_worker = rounds * S_K
    align = 8 * SC_SUBCORES
    n_pad = ((n + align - 1) // align) * align
    stripe = n_pad // SC_SUBCORES

    zstripe = jnp.zeros((stripe, 16), jnp.float32)
    mesh = plsc.VectorSubcoreMesh(core_axis_name="c", subcore_axis_name="s")

    @functools.partial(
        pl.kernel,
        mesh=mesh,
        out_type=jax.ShapeDtypeStruct((SC_CORES, n_pad, 16), jnp.float32),
        scratch_types=[
            pltpu.VMEM((2, S_K, S_PAD), jnp.int32),
            pltpu.VMEM((2, CHUNK + (S_PAD - S_BATCH), 16), jnp.float32),
            pltpu.VMEM_SHARED((n_pad, 16), jnp.float32),
            pltpu.SemaphoreType.DMA,
            pltpu.SemaphoreType.DMA,
            pltpu.SemaphoreType.DMA,
            pltpu.SemaphoreType.DMA,
        ],
        compiler_params=pltpu.CompilerParams(use_tc_tiling_on_sc=False),
    )
    def k(msg_hbm, recv_hbm, z_hbm, out_hbm, idx_v, msg_v, agg_sh,
          si0, sm0, si1, sm1):
        c = lax.axis_index("c")
        s = lax.axis_index("s")
        w = s * SC_CORES + c
        base = w * rows_per_worker
        sems = ((si0, sm0), (si1, sm1))

        def issue(rr, p):
            r0 = base + rr * S_K
            pltpu.async_copy(recv_hbm.at[pl.ds(row_off + r0, S_K)],
                             idx_v.at[p], sems[p][0])
            pltpu.async_copy(msg_hbm.at[pl.ds(r0 * S_BATCH, CHUNK)],
                             msg_v.at[p, pl.ds(0, CHUNK)], sems[p][1])

        def wait(p):
            pltpu.make_async_copy(recv_hbm.at[pl.ds(row_off, S_K)],
                                  idx_v.at[p], sems[p][0]).wait()
            pltpu.make_async_copy(msg_hbm.at[pl.ds(0, CHUNK)],
                                  msg_v.at[p, pl.ds(0, CHUNK)],
                                  sems[p][1]).wait()

        def scatter(p):
            for j in range(S_K):
                pltpu.sync_copy(
                    msg_v.at[p, pl.ds(j * S_BATCH, S_PAD)],
                    agg_sh.at[idx_v.at[p, j]],
                    add=True,
                )

        pltpu.sync_copy(z_hbm, agg_sh.at[pl.ds(s * stripe, stripe)])
        plsc.subcore_barrier()

        issue(0, 0)
        issue(1, 1)

        @pl.loop(0, rounds - 2, step=2)
        def _(r):
            wait(0)
            scatter(0)
            issue(r + 2, 0)
            wait(1)
            scatter(1)
            issue(r + 3, 1)

        wait(0)
        scatter(0)
        wait(1)
        scatter(1)

        plsc.subcore_barrier()
        pltpu.sync_copy(agg_sh.at[pl.ds(s * stripe, stripe)],
                        out_hbm.at[c, pl.ds(s * stripe, stripe)])

    return k(msg, recv2, zstripe)


def kernel(dyn, rel, send, recv,
           frel_W0, frel_b0, frel_W1, frel_b1, frel_W2, frel_b2, frel_W3,
           frel_b3, frel_W4, frel_b4,
           fdyn_W0, fdyn_b0, fdyn_W1, fdyn_b1, fdyn_W2, fdyn_b2, fdyn_W3,
           fdyn_b3, fdyn_W4, fdyn_b4):
    frel_Ws = [frel_W0, frel_W1, frel_W2, frel_W3, frel_W4]
    frel_bs = [frel_b0, frel_b1, frel_b2, frel_b3, frel_b4]
    fdyn_Ws = [fdyn_W0, fdyn_W1, fdyn_W2, fdyn_W3, fdyn_W4]
    fdyn_bs = [fdyn_b0, fdyn_b1, fdyn_b2, fdyn_b3, fdyn_b4]
    b, n, _ = dyn.shape
    e = rel.shape[1]
    e_ch = e // K_CH

    rel2 = rel.reshape(e, rel.shape[-1])
    align = 8 * SC_SUBCORES
    n_pad = ((n + align - 1) // align) * align
    recv2 = jnp.concatenate(
        [recv.reshape(e // S_BATCH, S_BATCH),
         jnp.full((e // S_BATCH, S_PAD - S_BATCH), n_pad - 1, jnp.int32)],
        axis=1)

    parts = []
    for kc in range(K_CH):
        msg_k = _edge_mlp_chunk(rel2, frel_Ws, frel_bs, e_ch,
                                kc * (e_ch // E_BLOCK))
        pk = _sc_scatter(msg_k, recv2, n, kc * (e_ch // S_BATCH))
        parts += [pk[0], pk[1]]

    out = _node_mlp(dyn.reshape(n, dyn.shape[-1]), parts, fdyn_Ws, fdyn_bs)
    return out.reshape(b, n, out.shape[-1])


# speedup vs baseline: 1.6542x; 1.4737x over previous
import functools

import jax
import jax.numpy as jnp
from jax import lax
from jax.experimental import pallas as pl
from jax.experimental.pallas import tpu as pltpu
from jax.experimental.pallas import tpu_sc as plsc

E_BLOCK = 8000
N_BLOCK = 2000
K_CH = 2

SC_CORES = 2
SC_SUBCORES = 16
NW = SC_CORES * SC_SUBCORES
S_BATCH = 125
S_PAD = 128
S_K = 4
CHUNK = S_BATCH * S_K


def _bdot(x, w):
    return jnp.dot(x, w, preferred_element_type=jnp.float32)


def _edge_body(x_ref, w0, b0, w1, b1, w2, b2, w3, b3, w4, b4, out_ref):
    x = x_ref[...]
    x = jnp.maximum(_bdot(x, w0[...]) + b0[...], 0.0)
    x = jnp.maximum(_bdot(x, w1[...]) + b1[...], 0.0)
    x = jnp.maximum(_bdot(x, w2[...]) + b2[...], 0.0)
    x = jnp.maximum(_bdot(x, w3[...]) + b3[...], 0.0)
    out_ref[:, 0:16] = _bdot(x, w4[...]) + b4[...]


def _full_spec(shape):
    return pl.BlockSpec(shape, lambda i: (0,) * len(shape))


def _edge_mlp_chunk(x, Ws, bs, e_ch, block_off):
    fin = x.shape[1]
    fout = Ws[-1].shape[1]
    specs = [pl.BlockSpec((E_BLOCK, fin), lambda i: (i + block_off, 0))]
    args = [x]
    for w, b in zip(Ws, bs):
        specs.append(_full_spec(w.shape))
        args.append(w)
        b2 = b.reshape(1, -1)
        specs.append(_full_spec(b2.shape))
        args.append(b2)
    return pl.pallas_call(
        _edge_body,
        grid=(e_ch // E_BLOCK,),
        in_specs=specs,
        out_specs=pl.BlockSpec((E_BLOCK, 128), lambda i: (i, 0)),
        out_shape=jax.ShapeDtypeStruct((e_ch, 128), jnp.float32),
    )(*args)


def _node_mlp(dyn, parts, Ws, bs):
    n, fdyn = dyn.shape
    fagg = parts[0].shape[1]
    fout = Ws[-1].shape[1]
    w0a = Ws[0][:fdyn]
    w0b = Ws[0][fdyn:]
    nparts = len(parts)

    def body(*refs):
        dyn_ref = refs[0]
        prefs = refs[1:1 + nparts]
        (w0a_r, w0b_r, b0_r, w1_r, b1_r, w2_r, b2_r, w3_r, b3_r, w4_r,
         b4_r) = refs[1 + nparts:-1]
        out_ref = refs[-1]
        p = prefs[0][...]
        for pr in prefs[1:]:
            p = p + pr[...]
        h = _bdot(dyn_ref[...], w0a_r[...]) + _bdot(p, w0b_r[...])
        h = jnp.maximum(h + b0_r[...], 0.0)
        h = jnp.maximum(_bdot(h, w1_r[...]) + b1_r[...], 0.0)
        h = jnp.maximum(_bdot(h, w2_r[...]) + b2_r[...], 0.0)
        h = jnp.maximum(_bdot(h, w3_r[...]) + b3_r[...], 0.0)
        out_ref[...] = _bdot(h, w4_r[...]) + b4_r[...]

    specs = [pl.BlockSpec((N_BLOCK, fdyn), lambda i: (i, 0))]
    args = [dyn]
    for p in parts:
        specs.append(pl.BlockSpec((N_BLOCK, fagg), lambda i: (i, 0)))
        args.append(p)
    specs += [_full_spec(w0a.shape), _full_spec(w0b.shape)]
    args += [w0a, w0b]
    b2 = bs[0].reshape(1, -1)
    specs.append(_full_spec(b2.shape))
    args.append(b2)
    for w, b in zip(Ws[1:], bs[1:]):
        specs.append(_full_spec(w.shape))
        args.append(w)
        b2 = b.reshape(1, -1)
        specs.append(_full_spec(b2.shape))
        args.append(b2)
    return pl.pallas_call(
        body,
        grid=(n // N_BLOCK,),
        in_specs=specs,
        out_specs=pl.BlockSpec((N_BLOCK, fout), lambda i: (i, 0)),
        out_shape=jax.ShapeDtypeStruct((n, fout), jnp.float32),
    )(*args)


def _sc_scatter(msg, recv2, n, row_off):
    e_ch = msg.shape[0]
    rounds = e_ch // (NW * CHUNK)
    assert rounds % 2 == 0 and rounds >= 4
    rows_per_worker = rounds * S_K
    align = 8 * SC_SUBCORES
    n_pad = ((n + align - 1) // align) * align
    stripe = n_pad // SC_SUBCORES

    zstripe = jnp.zeros((stripe, 16), jnp.float32)
    mesh = plsc.VectorSubcoreMesh(core_axis_name="c", subcore_axis_name="s")

    @functools.partial(
        pl.kernel,
        mesh=mesh,
        out_type=jax.ShapeDtypeStruct((SC_CORES, n_pad, 16), jnp.float32),
        scratch_types=[
            pltpu.VMEM((2, S_K, S_PAD), jnp.int32),
            pltpu.VMEM((2, CHUNK + (S_PAD - S_BATCH), 16), jnp.float32),
            pltpu.VMEM_SHARED((n_pad, 16), jnp.float32),
            pltpu.SemaphoreType.DMA,
            pltpu.SemaphoreType.DMA,
            pltpu.SemaphoreType.DMA,
            pltpu.SemaphoreType.DMA,
        ],
        compiler_params=pltpu.CompilerParams(use_tc_tiling_on_sc=False),
    )
    def k(msg_hbm, recv_hbm, z_hbm, out_hbm, idx_v, msg_v, agg_sh,
          si0, sm0, si1, sm1):
        c = lax.axis_index("c")
        s = lax.axis_index("s")
        w = s * SC_CORES + c
        base = w * rows_per_worker
        sems = ((si0, sm0), (si1, sm1))

        def issue(rr, p):
            r0 = base + rr * S_K
            pltpu.async_copy(recv_hbm.at[pl.ds(row_off + r0, S_K)],
                             idx_v.at[p], sems[p][0])
            pltpu.async_copy(msg_hbm.at[pl.ds(r0 * S_BATCH, CHUNK), pl.ds(0, 16)],
                             msg_v.at[p, pl.ds(0, CHUNK)], sems[p][1])

        def wait(p):
            pltpu.make_async_copy(recv_hbm.at[pl.ds(row_off, S_K)],
                                  idx_v.at[p], sems[p][0]).wait()
            pltpu.make_async_copy(msg_hbm.at[pl.ds(0, CHUNK), pl.ds(0, 16)],
                                  msg_v.at[p, pl.ds(0, CHUNK)],
                                  sems[p][1]).wait()

        def scatter(p):
            for j in range(S_K):
                pltpu.sync_copy(
                    msg_v.at[p, pl.ds(j * S_BATCH, S_PAD)],
                    agg_sh.at[idx_v.at[p, j]],
                    add=True,
                )

        pltpu.sync_copy(z_hbm, agg_sh.at[pl.ds(s * stripe, stripe)])
        plsc.subcore_barrier()

        issue(0, 0)
        issue(1, 1)

        @pl.loop(0, rounds - 2, step=2)
        def _(r):
            wait(0)
            scatter(0)
            issue(r + 2, 0)
            wait(1)
            scatter(1)
            issue(r + 3, 1)

        wait(0)
        scatter(0)
        wait(1)
        scatter(1)

        plsc.subcore_barrier()
        pltpu.sync_copy(agg_sh.at[pl.ds(s * stripe, stripe)],
                        out_hbm.at[c, pl.ds(s * stripe, stripe)])

    return k(msg, recv2, zstripe)


def kernel(dyn, rel, send, recv,
           frel_W0, frel_b0, frel_W1, frel_b1, frel_W2, frel_b2, frel_W3,
           frel_b3, frel_W4, frel_b4,
           fdyn_W0, fdyn_b0, fdyn_W1, fdyn_b1, fdyn_W2, fdyn_b2, fdyn_W3,
           fdyn_b3, fdyn_W4, fdyn_b4):
    frel_Ws = [frel_W0, frel_W1, frel_W2, frel_W3, frel_W4]
    frel_bs = [frel_b0, frel_b1, frel_b2, frel_b3, frel_b4]
    fdyn_Ws = [fdyn_W0, fdyn_W1, fdyn_W2, fdyn_W3, fdyn_W4]
    fdyn_bs = [fdyn_b0, fdyn_b1, fdyn_b2, fdyn_b3, fdyn_b4]
    b, n, _ = dyn.shape
    e = rel.shape[1]
    e_ch = e // K_CH

    rel2 = rel.reshape(e, rel.shape[-1])
    align = 8 * SC_SUBCORES
    n_pad = ((n + align - 1) // align) * align
    recv2 = jnp.concatenate(
        [recv.reshape(e // S_BATCH, S_BATCH),
         jnp.full((e // S_BATCH, S_PAD - S_BATCH), n_pad - 1, jnp.int32)],
        axis=1)

    parts = []
    for kc in range(K_CH):
        msg_k = _edge_mlp_chunk(rel2, frel_Ws, frel_bs, e_ch,
                                kc * (e_ch // E_BLOCK))
        pk = _sc_scatter(msg_k, recv2, n, kc * (e_ch // S_BATCH))
        parts += [pk[0], pk[1]]

    out = _node_mlp(dyn.reshape(n, dyn.shape[-1]), parts, fdyn_Ws, fdyn_bs)
    return out.reshape(b, n, out.shape[-1])


# speedup vs baseline: 1.6937x; 1.0239x over previous
import functools

import jax
import jax.numpy as jnp
from jax import lax
from jax.experimental import pallas as pl
from jax.experimental.pallas import tpu as pltpu
from jax.experimental.pallas import tpu_sc as plsc

E_BLOCK = 8000
N_BLOCK = 2000
K_CH = 2

SC_CORES = 2
SC_SUBCORES = 16
NW = SC_CORES * SC_SUBCORES
S_BATCH = 125
S_PAD = 128
S_K = 4
CHUNK = S_BATCH * S_K


def _bdot(x, w):
    return jnp.dot(x, w, preferred_element_type=jnp.float32)


def _edge_body(x_ref, w0, b0, w1, b1, w2, b2, w3, b3, w4, b4, out_ref):
    x = x_ref[...]
    x = jnp.maximum(_bdot(x, w0[...]) + b0[...], 0.0)
    x = jnp.maximum(_bdot(x, w1[...]) + b1[...], 0.0)
    x = jnp.maximum(_bdot(x, w2[...]) + b2[...], 0.0)
    x = jnp.maximum(_bdot(x, w3[...]) + b3[...], 0.0)
    out_ref[:, 0:16] = _bdot(x, w4[...]) + b4[...]


def _full_spec(shape):
    return pl.BlockSpec(shape, lambda i: (0,) * len(shape))


def _edge_mlp_chunk(x, Ws, bs, e_ch, block_off):
    fin = x.shape[1]
    fout = Ws[-1].shape[1]
    specs = [pl.BlockSpec((E_BLOCK, fin), lambda i: (i + block_off, 0))]
    args = [x]
    for w, b in zip(Ws, bs):
        specs.append(_full_spec(w.shape))
        args.append(w)
        b2 = b.reshape(1, -1)
        specs.append(_full_spec(b2.shape))
        args.append(b2)
    return pl.pallas_call(
        _edge_body,
        grid=(e_ch // E_BLOCK,),
        in_specs=specs,
        out_specs=pl.BlockSpec((E_BLOCK, 128), lambda i: (i, 0)),
        out_shape=jax.ShapeDtypeStruct((e_ch, 128), jnp.float32),
    )(*args)


def _node_mlp(dyn, parts, Ws, bs):
    n, fdyn = dyn.shape
    fagg = 16
    fout = Ws[-1].shape[1]
    w0a = Ws[0][:fdyn]
    w0b = Ws[0][fdyn:]
    nparts = len(parts)

    def body(*refs):
        dyn_ref = refs[0]
        prefs = refs[1:1 + nparts]
        (w0a_r, w0b_r, b0_r, w1_r, b1_r, w2_r, b2_r, w3_r, b3_r, w4_r,
         b4_r) = refs[1 + nparts:-1]
        out_ref = refs[-1]
        p = prefs[0][:, 0:fagg]
        for pr in prefs[1:]:
            p = p + pr[:, 0:fagg]
        h = _bdot(dyn_ref[...], w0a_r[...]) + _bdot(p, w0b_r[...])
        h = jnp.maximum(h + b0_r[...], 0.0)
        h = jnp.maximum(_bdot(h, w1_r[...]) + b1_r[...], 0.0)
        h = jnp.maximum(_bdot(h, w2_r[...]) + b2_r[...], 0.0)
        h = jnp.maximum(_bdot(h, w3_r[...]) + b3_r[...], 0.0)
        out_ref[...] = _bdot(h, w4_r[...]) + b4_r[...]

    specs = [pl.BlockSpec((N_BLOCK, fdyn), lambda i: (i, 0))]
    args = [dyn]
    for p in parts:
        specs.append(pl.BlockSpec((N_BLOCK, 128), lambda i: (i, 0)))
        args.append(p)
    specs += [_full_spec(w0a.shape), _full_spec(w0b.shape)]
    args += [w0a, w0b]
    b2 = bs[0].reshape(1, -1)
    specs.append(_full_spec(b2.shape))
    args.append(b2)
    for w, b in zip(Ws[1:], bs[1:]):
        specs.append(_full_spec(w.shape))
        args.append(w)
        b2 = b.reshape(1, -1)
        specs.append(_full_spec(b2.shape))
        args.append(b2)
    return pl.pallas_call(
        body,
        grid=(n // N_BLOCK,),
        in_specs=specs,
        out_specs=pl.BlockSpec((N_BLOCK, fout), lambda i: (i, 0)),
        out_shape=jax.ShapeDtypeStruct((n, fout), jnp.float32),
    )(*args)


def _sc_scatter(msg, recv2, n, row_off):
    e_ch = msg.shape[0]
    rounds = e_ch // (NW * CHUNK)
    assert rounds % 2 == 0 and rounds >= 4
    rows_per_worker = rounds * S_K
    align = 8 * SC_SUBCORES
    n_pad = ((n + align - 1) // align) * align
    stripe = n_pad // SC_SUBCORES

    zstripe = jnp.zeros((stripe, 16), jnp.float32)
    mesh = plsc.VectorSubcoreMesh(core_axis_name="c", subcore_axis_name="s")

    @functools.partial(
        pl.kernel,
        mesh=mesh,
        out_type=jax.ShapeDtypeStruct((SC_CORES, n_pad, 128), jnp.float32),
        scratch_types=[
            pltpu.VMEM((2, S_K, S_PAD), jnp.int32),
            pltpu.VMEM((2, CHUNK + (S_PAD - S_BATCH), 16), jnp.float32),
            pltpu.VMEM_SHARED((n_pad, 16), jnp.float32),
            pltpu.SemaphoreType.DMA,
            pltpu.SemaphoreType.DMA,
            pltpu.SemaphoreType.DMA,
            pltpu.SemaphoreType.DMA,
        ],
        compiler_params=pltpu.CompilerParams(use_tc_tiling_on_sc=False),
    )
    def k(msg_hbm, recv_hbm, z_hbm, out_hbm, idx_v, msg_v, agg_sh,
          si0, sm0, si1, sm1):
        c = lax.axis_index("c")
        s = lax.axis_index("s")
        w = s * SC_CORES + c
        base = w * rows_per_worker
        sems = ((si0, sm0), (si1, sm1))

        def issue(rr, p):
            r0 = base + rr * S_K
            pltpu.async_copy(recv_hbm.at[pl.ds(row_off + r0, S_K)],
                             idx_v.at[p], sems[p][0])
            pltpu.async_copy(msg_hbm.at[pl.ds(r0 * S_BATCH, CHUNK), pl.ds(0, 16)],
                             msg_v.at[p, pl.ds(0, CHUNK)], sems[p][1])

        def wait(p):
            pltpu.make_async_copy(recv_hbm.at[pl.ds(row_off, S_K)],
                                  idx_v.at[p], sems[p][0]).wait()
            pltpu.make_async_copy(msg_hbm.at[pl.ds(0, CHUNK), pl.ds(0, 16)],
                                  msg_v.at[p, pl.ds(0, CHUNK)],
                                  sems[p][1]).wait()

        def scatter(p):
            for j in range(S_K):
                pltpu.sync_copy(
                    msg_v.at[p, pl.ds(j * S_BATCH, S_PAD)],
                    agg_sh.at[idx_v.at[p, j]],
                    add=True,
                )

        pltpu.sync_copy(z_hbm, agg_sh.at[pl.ds(s * stripe, stripe)])
        plsc.subcore_barrier()

        issue(0, 0)
        issue(1, 1)

        @pl.loop(0, rounds - 2, step=2)
        def _(r):
            wait(0)
            scatter(0)
            issue(r + 2, 0)
            wait(1)
            scatter(1)
            issue(r + 3, 1)

        wait(0)
        scatter(0)
        wait(1)
        scatter(1)

        plsc.subcore_barrier()
        pltpu.sync_copy(agg_sh.at[pl.ds(s * stripe, stripe)],
                        out_hbm.at[c, pl.ds(s * stripe, stripe), pl.ds(0, 16)])

    return k(msg, recv2, zstripe)


def kernel(dyn, rel, send, recv,
           frel_W0, frel_b0, frel_W1, frel_b1, frel_W2, frel_b2, frel_W3,
           frel_b3, frel_W4, frel_b4,
           fdyn_W0, fdyn_b0, fdyn_W1, fdyn_b1, fdyn_W2, fdyn_b2, fdyn_W3,
           fdyn_b3, fdyn_W4, fdyn_b4):
    frel_Ws = [frel_W0, frel_W1, frel_W2, frel_W3, frel_W4]
    frel_bs = [frel_b0, frel_b1, frel_b2, frel_b3, frel_b4]
    fdyn_Ws = [fdyn_W0, fdyn_W1, fdyn_W2, fdyn_W3, fdyn_W4]
    fdyn_bs = [fdyn_b0, fdyn_b1, fdyn_b2, fdyn_b3, fdyn_b4]
    b, n, _ = dyn.shape
    e = rel.shape[1]
    e_ch = e // K_CH

    rel2 = rel.reshape(e, rel.shape[-1])
    align = 8 * SC_SUBCORES
    n_pad = ((n + align - 1) // align) * align
    recv2 = jnp.concatenate(
        [recv.reshape(e // S_BATCH, S_BATCH),
         jnp.full((e // S_BATCH, S_PAD - S_BATCH), n_pad - 1, jnp.int32)],
        axis=1)

    parts = []
    for kc in range(K_CH):
        msg_k = _edge_mlp_chunk(rel2, frel_Ws, frel_bs, e_ch,
                                kc * (e_ch // E_BLOCK))
        pk = _sc_scatter(msg_k, recv2, n, kc * (e_ch // S_BATCH))
        parts += [pk[0], pk[1]]

    out = _node_mlp(dyn.reshape(n, dyn.shape[-1]), parts, fdyn_Ws, fdyn_bs)
    return out.reshape(b, n, out.shape[-1])
